# baseline (speedup 1.0000x reference)
import functools

import jax
import jax.numpy as jnp
from jax import lax
from jax.experimental import pallas as pl
from jax.experimental.pallas import tpu as pltpu
from jax.experimental.pallas import tpu_sc as plsc

_N = 10000
_H = 256
_E = 160000
_HH = 128
_LANES = 16
_NC = 2
_NT = 16
_EPT = _E // _NT
_K = 80
_NCH = _EPT // _K
_NZ = 624
_NREM = _N - _NT * _NZ
_R = 1000
_G = _N // _R


def _pre_body(x_ref, w_ref, b_ref, a_ref, bb_ref):
    xb = x_ref[...]
    w = w_ref[...]
    a_ref[...] = (
        jnp.dot(xb, w[:_H, :], preferred_element_type=jnp.float32) + b_ref[...]
    )
    bb_ref[...] = jnp.dot(xb, w[_H:, :], preferred_element_type=jnp.float32)


def _pre_projections(x, We1_i, be1_row):
    return pl.pallas_call(
        _pre_body,
        grid=(_G, _NC),
        in_specs=[
            pl.BlockSpec((_R, _H), lambda i, h: (i, 0)),
            pl.BlockSpec((2 * _H, _HH), lambda i, h: (0, h)),
            pl.BlockSpec((1, _HH), lambda i, h: (0, h)),
        ],
        out_specs=[
            pl.BlockSpec((_R, _HH), lambda i, h: (h * _G + i, 0)),
            pl.BlockSpec((_R, _HH), lambda i, h: (h * _G + i, 0)),
        ],
        out_shape=[jax.ShapeDtypeStruct((_NC * _N, _HH), jnp.float32)] * 2,
    )(x, We1_i, be1_row)


def _post_body(s0_ref, s1_ref, x_ref, cnt_ref, we2_ref, wn1_ref, wn2_ref,
               be2_ref, bn1_ref, bn2_ref, g_ref, bt_ref, out_ref):
    s = jnp.concatenate([s0_ref[...], s1_ref[...]], axis=1)
    agg = (
        jnp.dot(s, we2_ref[...], preferred_element_type=jnp.float32)
        + cnt_ref[...] * be2_ref[...]
    )
    xb = x_ref[...]
    wn1 = wn1_ref[...]
    h1 = jnp.maximum(
        jnp.dot(xb, wn1[:_H, :], preferred_element_type=jnp.float32)
        + jnp.dot(agg, wn1[_H:, :], preferred_element_type=jnp.float32)
        + bn1_ref[...],
        0.0,
    )
    u = jnp.dot(h1, wn2_ref[...], preferred_element_type=jnp.float32) + bn2_ref[...]
    y = u + xb
    mu = jnp.mean(y, axis=1, keepdims=True)
    var = jnp.mean((y - mu) ** 2, axis=1, keepdims=True)
    out_ref[...] = (y - mu) * lax.rsqrt(var + 1e-5) * g_ref[...] + bt_ref[...]


def _post_update(s0, s1, x, counts, We2_i, Wn1_i, Wn2_i, be2_row, bn1_row,
                 bn2_row, g_row, bt_row):
    full = lambda shape: pl.BlockSpec(shape, lambda i: (0, 0))
    return pl.pallas_call(
        _post_body,
        grid=(_G,),
        in_specs=[
            pl.BlockSpec((_R, _HH), lambda i: (i, 0)),
            pl.BlockSpec((_R, _HH), lambda i: (i, 0)),
            pl.BlockSpec((_R, _H), lambda i: (i, 0)),
            pl.BlockSpec((_R, 1), lambda i: (i, 0)),
            full((_H, _H)),
            full((2 * _H, _H)),
            full((_H, _H)),
            full((1, _H)),
            full((1, _H)),
            full((1, _H)),
            full((1, _H)),
            full((1, _H)),
        ],
        out_specs=pl.BlockSpec((_R, _H), lambda i: (i, 0)),
        out_shape=jax.ShapeDtypeStruct((_N, _H), jnp.float32),
    )(s0, s1, x, counts, We2_i, Wn1_i, Wn2_i, be2_row, bn1_row, bn2_row,
      g_row, bt_row)


def _make_sc_edge_kernel():
    mesh = plsc.VectorSubcoreMesh(core_axis_name="c", subcore_axis_name="s")

    @functools.partial(
        pl.kernel,
        mesh=mesh,
        out_type=jax.ShapeDtypeStruct((_NC, _N, _HH), jnp.float32),
        scratch_types=[
            pltpu.VMEM((_K,), jnp.int32),
            pltpu.VMEM((_K,), jnp.int32),
            pltpu.VMEM((_K,), jnp.int32),
            pltpu.VMEM((_K,), jnp.int32),
            pltpu.VMEM((_K,), jnp.int32),
            pltpu.VMEM((_K,), jnp.int32),
            pltpu.VMEM((_K, _HH), jnp.float32),
            pltpu.VMEM((_K, _HH), jnp.float32),
            pltpu.SemaphoreType.DMA,
            pltpu.SemaphoreType.DMA,
            pltpu.VMEM_SHARED((_N, _HH), jnp.float32),
        ],
    )
    def sc_edge(a_h, b_h, src_h, dst_h, out_h,
                ibs0, ibs1, ibd0, ibd1, dg0, dg1, r0, r1,
                sa0, sa1, s_sh):
        c = lax.axis_index("c")
        t = lax.axis_index("s")
        ibs = (ibs0, ibs1)
        ibd = (ibd0, ibd1)
        dg = (dg0, dg1)
        r = (r0, r1)
        sa = (sa0, sa1)

        def zrow(q, carry):
            for u in range(_HH // _LANES):
                r0[q, pl.ds(u * _LANES, _LANES)] = jnp.zeros(
                    (_LANES,), jnp.float32)
            return carry
        lax.fori_loop(0, _K, zrow, 0)
        for q in range(_NZ // _K):
            pltpu.sync_copy(r0, s_sh.at[pl.ds(t * _NZ + q * _K, _K)])
        zrem = _NZ - (_NZ // _K) * _K
        if zrem:
            pltpu.sync_copy(
                r0.at[pl.ds(0, zrem)],
                s_sh.at[pl.ds(t * _NZ + (_NZ // _K) * _K, zrem)])

        @pl.when(t == _NT - 1)
        def _zero_rem():
            pltpu.sync_copy(r0.at[pl.ds(0, _NREM)],
                            s_sh.at[pl.ds(_NT * _NZ, _NREM)])
        plsc.subcore_barrier()

        cn = jnp.broadcast_to((c * _N).astype(jnp.int32), (_LANES,))
        ebase = t * _EPT

        def load_bias(p, j):
            pltpu.sync_copy(src_h.at[pl.ds(ebase + j * _K, _K)], ibs[p])
            pltpu.sync_copy(dst_h.at[pl.ds(ebase + j * _K, _K)], ibd[p])
            for u in range(_K // _LANES):
                sl = pl.ds(u * _LANES, _LANES)
                ibs[p][sl] = ibs[p][sl] + cn
                dg[p][sl] = ibd[p][sl] + cn

        def start_a(p):
            pltpu.async_copy(a_h.at[ibs[p]], r[p], sa[p])

        def wait_a(p):
            pltpu.make_async_copy(a_h.at[ibs[p]], r[p], sa[p]).wait()

        def relu(p):
            def relu_row(q, inner):
                for u in range(_HH // _LANES):
                    sl = pl.ds(u * _LANES, _LANES)
                    r[p][q, sl] = jnp.maximum(r[p][q, sl], 0.0)
                return inner
            lax.fori_loop(0, _K, relu_row, 0)

        def scatter(p):
            pltpu.sync_copy(r[p], s_sh.at[ibd[p]], add=True)

        load_bias(0, 0)
        start_a(0)
        wait_a(0)
        pltpu.sync_copy(b_h.at[dg[0]], r[0], add=True)
        relu(0)
        scatter(0)
        load_bias(0, 1)
        start_a(0)
        wait_a(0)

        def pair(m, carry):
            load_bias(1, 2 * m + 2)
            start_a(1)
            pltpu.sync_copy(b_h.at[dg[0]], r[0], add=True)
            relu(0)
            wait_a(1)
            scatter(0)
            jn = jnp.minimum(2 * m + 3, _NCH - 1)
            load_bias(0, jn)
            start_a(0)
            pltpu.sync_copy(b_h.at[dg[1]], r[1], add=True)
            relu(1)
            wait_a(0)
            scatter(1)
            return carry
        lax.fori_loop(0, (_NCH - 1) // 2, pair, 0)

        plsc.subcore_barrier()
        pltpu.sync_copy(s_sh.at[pl.ds(t * _NZ, _NZ)],
                        out_h.at[c, pl.ds(t * _NZ, _NZ)])

        @pl.when(t == _NT - 1)
        def _copy_rem():
            pltpu.sync_copy(s_sh.at[pl.ds(_NT * _NZ, _NREM)],
                            out_h.at[c, pl.ds(_NT * _NZ, _NREM)])

    return sc_edge


_sc_edge = _make_sc_edge_kernel()


def kernel(x, We1, be1, We2, be2, Wn1, bn1, Wn2, bn2, gamma, beta, edge_index):
    L = We1.shape[0]
    assert x.shape == (_N, _H) and edge_index.shape == (2, _E)

    src = edge_index[0]
    dst = edge_index[1]


    counts = jnp.bincount(dst, length=_N).astype(jnp.float32).reshape(_N, 1)

    row = lambda v: v.reshape(1, _H)
    for i in range(L):
        a_fl, b_fl = _pre_projections(x, We1[i], row(be1[i]))
        s = _sc_edge(a_fl, b_fl, src, dst)
        x = _post_update(s[0], s[1], x, counts, We2[i], Wn1[i], Wn2[i],
                         row(be2[i]), row(bn1[i]), row(bn2[i]),
                         row(gamma[i]), row(beta[i]))
    return x


# speedup vs baseline: 1.3074x; 1.3074x over previous
import functools

import jax
import jax.numpy as jnp
from jax import lax
from jax.experimental import pallas as pl
from jax.experimental.pallas import tpu as pltpu
from jax.experimental.pallas import tpu_sc as plsc

_N = 10000
_H = 256
_E = 160000
_HH = 128
_LANES = 16
_NC = 2
_NT = 16
_EPT = _E // _NT
_K = 80
_NCH = _EPT // _K
_NZ = 624
_NREM = _N - _NT * _NZ
_R = 1000
_G = _N // _R


def _pre_body(x_ref, w_ref, b_ref, a_ref, bb_ref):
    xb = x_ref[...]
    w = w_ref[...]
    a_ref[...] = (
        jnp.dot(xb, w[:_H, :], preferred_element_type=jnp.float32) + b_ref[...]
    )
    bb_ref[...] = jnp.dot(xb, w[_H:, :], preferred_element_type=jnp.float32)


def _pre_projections(x, We1_i, be1_row):
    return pl.pallas_call(
        _pre_body,
        grid=(_G, _NC),
        in_specs=[
            pl.BlockSpec((_R, _H), lambda i, h: (i, 0)),
            pl.BlockSpec((2 * _H, _HH), lambda i, h: (0, h)),
            pl.BlockSpec((1, _HH), lambda i, h: (0, h)),
        ],
        out_specs=[
            pl.BlockSpec((_R, _HH), lambda i, h: (h * _G + i, 0)),
            pl.BlockSpec((_R, _HH), lambda i, h: (h * _G + i, 0)),
        ],
        out_shape=[jax.ShapeDtypeStruct((_NC * _N, _HH), jnp.float32)] * 2,
    )(x, We1_i, be1_row)


def _post_body(s0_ref, s1_ref, x_ref, cnt_ref, we2_ref, wn1_ref, wn2_ref,
               be2_ref, bn1_ref, bn2_ref, g_ref, bt_ref, out_ref):
    s = jnp.concatenate([s0_ref[...], s1_ref[...]], axis=1)
    agg = (
        jnp.dot(s, we2_ref[...], preferred_element_type=jnp.float32)
        + cnt_ref[...] * be2_ref[...]
    )
    xb = x_ref[...]
    wn1 = wn1_ref[...]
    h1 = jnp.maximum(
        jnp.dot(xb, wn1[:_H, :], preferred_element_type=jnp.float32)
        + jnp.dot(agg, wn1[_H:, :], preferred_element_type=jnp.float32)
        + bn1_ref[...],
        0.0,
    )
    u = jnp.dot(h1, wn2_ref[...], preferred_element_type=jnp.float32) + bn2_ref[...]
    y = u + xb
    mu = jnp.mean(y, axis=1, keepdims=True)
    var = jnp.mean((y - mu) ** 2, axis=1, keepdims=True)
    out_ref[...] = (y - mu) * lax.rsqrt(var + 1e-5) * g_ref[...] + bt_ref[...]


def _post_update(s0, s1, x, counts, We2_i, Wn1_i, Wn2_i, be2_row, bn1_row,
                 bn2_row, g_row, bt_row):
    full = lambda shape: pl.BlockSpec(shape, lambda i: (0, 0))
    return pl.pallas_call(
        _post_body,
        grid=(_G,),
        in_specs=[
            pl.BlockSpec((_R, _HH), lambda i: (i, 0)),
            pl.BlockSpec((_R, _HH), lambda i: (i, 0)),
            pl.BlockSpec((_R, _H), lambda i: (i, 0)),
            pl.BlockSpec((_R, 1), lambda i: (i, 0)),
            full((_H, _H)),
            full((2 * _H, _H)),
            full((_H, _H)),
            full((1, _H)),
            full((1, _H)),
            full((1, _H)),
            full((1, _H)),
            full((1, _H)),
        ],
        out_specs=pl.BlockSpec((_R, _H), lambda i: (i, 0)),
        out_shape=jax.ShapeDtypeStruct((_N, _H), jnp.float32),
    )(s0, s1, x, counts, We2_i, Wn1_i, Wn2_i, be2_row, bn1_row, bn2_row,
      g_row, bt_row)


def _make_sc_edge_kernel():
    mesh = plsc.VectorSubcoreMesh(core_axis_name="c", subcore_axis_name="s")

    @functools.partial(
        pl.kernel,
        mesh=mesh,
        out_type=jax.ShapeDtypeStruct((_NC, _N, _HH), jnp.float32),
        scratch_types=[
            pltpu.VMEM((_K,), jnp.int32),
            pltpu.VMEM((_K,), jnp.int32),
            pltpu.VMEM((_K,), jnp.int32),
            pltpu.VMEM((_K,), jnp.int32),
            pltpu.VMEM((_K,), jnp.int32),
            pltpu.VMEM((_K,), jnp.int32),
            pltpu.VMEM((_K,), jnp.int32),
            pltpu.VMEM((_K,), jnp.int32),
            pltpu.VMEM((_K, _HH), jnp.float32),
            pltpu.VMEM((_K, _HH), jnp.float32),
            pltpu.SemaphoreType.DMA,
            pltpu.SemaphoreType.DMA,
            pltpu.SemaphoreType.DMA,
            pltpu.SemaphoreType.DMA,
            pltpu.VMEM_SHARED((_N, _HH), jnp.float32),
        ],
    )
    def sc_edge(a_h, b_h, src_h, dst_h, out_h,
                ibs0, ibs1, ibd0, ibd1, dg0, dg1, sc0, sc1, r0, r1,
                sa0, sa1, si0, si1, s_sh):
        c = lax.axis_index("c")
        t = lax.axis_index("s")
        ibs = (ibs0, ibs1)
        ibd = (ibd0, ibd1)
        dg = (dg0, dg1)
        scat = (sc0, sc1)
        r = (r0, r1)
        sa = (sa0, sa1)
        si = (si0, si1)

        def zrow(q, carry):
            for u in range(_HH // _LANES):
                r0[q, pl.ds(u * _LANES, _LANES)] = jnp.zeros(
                    (_LANES,), jnp.float32)
            return carry
        lax.fori_loop(0, _K, zrow, 0)
        for q in range(_NZ // _K):
            pltpu.sync_copy(r0, s_sh.at[pl.ds(t * _NZ + q * _K, _K)])
        zrem = _NZ - (_NZ // _K) * _K
        if zrem:
            pltpu.sync_copy(
                r0.at[pl.ds(0, zrem)],
                s_sh.at[pl.ds(t * _NZ + (_NZ // _K) * _K, zrem)])

        @pl.when(t == _NT - 1)
        def _zero_rem():
            pltpu.sync_copy(r0.at[pl.ds(0, _NREM)],
                            s_sh.at[pl.ds(_NT * _NZ, _NREM)])
        plsc.subcore_barrier()

        cn = jnp.broadcast_to((c * _N).astype(jnp.int32), (_LANES,))
        ebase = t * _EPT

        def start_idx(p, j):
            off = pl.ds(ebase + j * _K, _K)
            pltpu.async_copy(src_h.at[off], ibs[p], si[p])
            pltpu.async_copy(dst_h.at[off], ibd[p], si[p])

        def wait_idx(p, j):
            off = pl.ds(ebase + j * _K, _K)
            pltpu.make_async_copy(src_h.at[off], ibs[p], si[p]).wait()
            pltpu.make_async_copy(dst_h.at[off], ibd[p], si[p]).wait()

        def bias(p):
            for u in range(_K // _LANES):
                sl = pl.ds(u * _LANES, _LANES)
                ibs[p][sl] = ibs[p][sl] + cn
                dg[p][sl] = ibd[p][sl] + cn
                scat[p][sl] = ibd[p][sl]

        def start_a(p):
            pltpu.async_copy(a_h.at[ibs[p]], r[p], sa[p])

        def wait_a(p):
            pltpu.make_async_copy(a_h.at[ibs[p]], r[p], sa[p]).wait()

        def relu(p):
            def relu_row(q, inner):
                for u in range(_HH // _LANES):
                    sl = pl.ds(u * _LANES, _LANES)
                    r[p][q, sl] = jnp.maximum(r[p][q, sl], 0.0)
                return inner
            lax.fori_loop(0, _K, relu_row, 0)

        def scatter(p):
            pltpu.sync_copy(r[p], s_sh.at[scat[p]], add=True)

        def process(p, j, jnext):
            q = 1 - p
            bias(q)
            start_a(q)
            start_idx(p, jnext)
            pltpu.sync_copy(b_h.at[dg[p]], r[p], add=True)
            relu(p)
            wait_a(q)
            wait_idx(p, jnext)
            scatter(p)

        pltpu.sync_copy(src_h.at[pl.ds(ebase, _K)], ibs[0])
        pltpu.sync_copy(dst_h.at[pl.ds(ebase, _K)], ibd[0])
        bias(0)
        start_a(0)
        pltpu.sync_copy(src_h.at[pl.ds(ebase + _K, _K)], ibs[1])
        pltpu.sync_copy(dst_h.at[pl.ds(ebase + _K, _K)], ibd[1])
        wait_a(0)

        last = _NCH - 1

        def pair(m, carry):
            j = 2 * m
            process(0, j, jnp.minimum(j + 2, last))
            process(1, j + 1, jnp.minimum(j + 3, last))
            return carry
        lax.fori_loop(0, _NCH // 2, pair, 0)
        process(0, last, last)

        plsc.subcore_barrier()
        pltpu.sync_copy(s_sh.at[pl.ds(t * _NZ, _NZ)],
                        out_h.at[c, pl.ds(t * _NZ, _NZ)])

        @pl.when(t == _NT - 1)
        def _copy_rem():
            pltpu.sync_copy(s_sh.at[pl.ds(_NT * _NZ, _NREM)],
                            out_h.at[c, pl.ds(_NT * _NZ, _NREM)])

    return sc_edge


_sc_edge = _make_sc_edge_kernel()


def kernel(x, We1, be1, We2, be2, Wn1, bn1, Wn2, bn2, gamma, beta, edge_index):
    L = We1.shape[0]
    assert x.shape == (_N, _H) and edge_index.shape == (2, _E)

    src = edge_index[0]
    dst = edge_index[1]


    counts = jnp.bincount(dst, length=_N).astype(jnp.float32).reshape(_N, 1)

    row = lambda v: v.reshape(1, _H)
    for i in range(L):
        a_fl, b_fl = _pre_projections(x, We1[i], row(be1[i]))
        s = _sc_edge(a_fl, b_fl, src, dst)
        x = _post_update(s[0], s[1], x, counts, We2[i], Wn1[i], Wn2[i],
                         row(be2[i]), row(bn1[i]), row(bn2[i]),
                         row(gamma[i]), row(beta[i]))
    return x


# speedup vs baseline: 1.6527x; 1.2642x over previous
import functools

import jax
import jax.numpy as jnp
from jax import lax
from jax.experimental import pallas as pl
from jax.experimental.pallas import tpu as pltpu
from jax.experimental.pallas import tpu_sc as plsc

_N = 10000
_H = 256
_E = 160000
_HH = 128
_LANES = 16
_NC = 2
_NT = 16
_EPT = _E // _NT
_K = 80
_NCH = _EPT // _K
_NZ = 624
_NREM = _N - _NT * _NZ
_R = 1000
_G = _N // _R


def _pre_body(x_ref, w_ref, b_ref, a_ref, bb_ref):
    xb = x_ref[...]
    w = w_ref[...]
    a_ref[...] = (
        jnp.dot(xb, w[:_H, :], preferred_element_type=jnp.float32) + b_ref[...]
    )
    bb_ref[...] = jnp.dot(xb, w[_H:, :], preferred_element_type=jnp.float32)


def _pre_projections(x, We1_i, be1_row):
    return pl.pallas_call(
        _pre_body,
        grid=(_G, _NC),
        in_specs=[
            pl.BlockSpec((_R, _H), lambda i, h: (i, 0)),
            pl.BlockSpec((2 * _H, _HH), lambda i, h: (0, h)),
            pl.BlockSpec((1, _HH), lambda i, h: (0, h)),
        ],
        out_specs=[
            pl.BlockSpec((_R, _HH), lambda i, h: (h * _G + i, 0)),
            pl.BlockSpec((_R, _HH), lambda i, h: (h * _G + i, 0)),
        ],
        out_shape=[jax.ShapeDtypeStruct((_NC * _N, _HH), jnp.float32)] * 2,
    )(x, We1_i, be1_row)


def _post_body(s0_ref, s1_ref, x_ref, cnt_ref, we2_ref, wn1_ref, wn2_ref,
               be2_ref, bn1_ref, bn2_ref, g_ref, bt_ref, out_ref):
    s = jnp.concatenate([s0_ref[...], s1_ref[...]], axis=1)
    agg = (
        jnp.dot(s, we2_ref[...], preferred_element_type=jnp.float32)
        + cnt_ref[...] * be2_ref[...]
    )
    xb = x_ref[...]
    wn1 = wn1_ref[...]
    h1 = jnp.maximum(
        jnp.dot(xb, wn1[:_H, :], preferred_element_type=jnp.float32)
        + jnp.dot(agg, wn1[_H:, :], preferred_element_type=jnp.float32)
        + bn1_ref[...],
        0.0,
    )
    u = jnp.dot(h1, wn2_ref[...], preferred_element_type=jnp.float32) + bn2_ref[...]
    y = u + xb
    mu = jnp.mean(y, axis=1, keepdims=True)
    var = jnp.mean((y - mu) ** 2, axis=1, keepdims=True)
    out_ref[...] = (y - mu) * lax.rsqrt(var + 1e-5) * g_ref[...] + bt_ref[...]


def _post_update(s0, s1, x, counts, We2_i, Wn1_i, Wn2_i, be2_row, bn1_row,
                 bn2_row, g_row, bt_row):
    full = lambda shape: pl.BlockSpec(shape, lambda i: (0, 0))
    return pl.pallas_call(
        _post_body,
        grid=(_G,),
        in_specs=[
            pl.BlockSpec((_R, _HH), lambda i: (i, 0)),
            pl.BlockSpec((_R, _HH), lambda i: (i, 0)),
            pl.BlockSpec((_R, _H), lambda i: (i, 0)),
            pl.BlockSpec((_R, 1), lambda i: (i, 0)),
            full((_H, _H)),
            full((2 * _H, _H)),
            full((_H, _H)),
            full((1, _H)),
            full((1, _H)),
            full((1, _H)),
            full((1, _H)),
            full((1, _H)),
        ],
        out_specs=pl.BlockSpec((_R, _H), lambda i: (i, 0)),
        out_shape=jax.ShapeDtypeStruct((_N, _H), jnp.float32),
    )(s0, s1, x, counts, We2_i, Wn1_i, Wn2_i, be2_row, bn1_row, bn2_row,
      g_row, bt_row)


def _make_sc_edge_kernel():
    mesh = plsc.VectorSubcoreMesh(core_axis_name="c", subcore_axis_name="s")

    @functools.partial(
        pl.kernel,
        mesh=mesh,
        out_type=jax.ShapeDtypeStruct((_NC, _N, _HH), jnp.float32),
        scratch_types=[
            pltpu.VMEM((_K,), jnp.int32),
            pltpu.VMEM((_K,), jnp.int32),
            pltpu.VMEM((_K,), jnp.int32),
            pltpu.VMEM((_K,), jnp.int32),
            pltpu.VMEM((_K,), jnp.int32),
            pltpu.VMEM((_K,), jnp.int32),
            pltpu.VMEM((_K,), jnp.int32),
            pltpu.VMEM((_K,), jnp.int32),
            pltpu.VMEM((_K, _HH), jnp.float32),
            pltpu.VMEM((_K, _HH), jnp.float32),
            pltpu.SemaphoreType.DMA,
            pltpu.SemaphoreType.DMA,
            pltpu.SemaphoreType.DMA,
            pltpu.SemaphoreType.DMA,
            pltpu.SemaphoreType.DMA,
            pltpu.SemaphoreType.DMA,
            pltpu.VMEM_SHARED((_N, _HH), jnp.float32),
        ],
    )
    def sc_edge(a_h, b_h, src_h, dst_h, out_h,
                ibs0, ibs1, ibd0, ibd1, dg0, dg1, sc0, sc1, r0, r1,
                sa0, sa1, si0, si1, sb0, sb1, s_sh):
        c = lax.axis_index("c")
        t = lax.axis_index("s")
        ibs = (ibs0, ibs1)
        ibd = (ibd0, ibd1)
        dg = (dg0, dg1)
        scat = (sc0, sc1)
        r = (r0, r1)
        sa = (sa0, sa1)
        si = (si0, si1)
        sb = (sb0, sb1)

        def zrow(q, carry):
            for u in range(_HH // _LANES):
                r0[q, pl.ds(u * _LANES, _LANES)] = jnp.zeros(
                    (_LANES,), jnp.float32)
            return carry
        lax.fori_loop(0, _K, zrow, 0)
        for q in range(_NZ // _K):
            pltpu.sync_copy(r0, s_sh.at[pl.ds(t * _NZ + q * _K, _K)])
        zrem = _NZ - (_NZ // _K) * _K
        if zrem:
            pltpu.sync_copy(
                r0.at[pl.ds(0, zrem)],
                s_sh.at[pl.ds(t * _NZ + (_NZ // _K) * _K, zrem)])

        @pl.when(t == _NT - 1)
        def _zero_rem():
            pltpu.sync_copy(r0.at[pl.ds(0, _NREM)],
                            s_sh.at[pl.ds(_NT * _NZ, _NREM)])
        plsc.subcore_barrier()

        cn = jnp.broadcast_to((c * _N).astype(jnp.int32), (_LANES,))
        ebase = t * _EPT

        def start_idx(p, j):
            off = pl.ds(ebase + j * _K, _K)
            pltpu.async_copy(src_h.at[off], ibs[p], si[p])
            pltpu.async_copy(dst_h.at[off], ibd[p], si[p])

        def wait_idx(p, j):
            off = pl.ds(ebase + j * _K, _K)
            pltpu.make_async_copy(src_h.at[off], ibs[p], si[p]).wait()
            pltpu.make_async_copy(dst_h.at[off], ibd[p], si[p]).wait()

        def bias(p):
            for u in range(_K // _LANES):
                sl = pl.ds(u * _LANES, _LANES)
                ibs[p][sl] = ibs[p][sl] + cn
                dg[p][sl] = ibd[p][sl] + cn
                scat[p][sl] = ibd[p][sl]

        def start_a(p):
            pltpu.async_copy(a_h.at[ibs[p]], r[p], sa[p])

        def wait_a(p):
            pltpu.make_async_copy(a_h.at[ibs[p]], r[p], sa[p]).wait()

        def start_b(p):
            pltpu.async_copy(b_h.at[dg[p]], r[p], sb[p], add=True)

        def wait_b(p):
            pltpu.make_async_copy(b_h.at[dg[p]], r[p], sb[p]).wait()

        def relu(p):
            def relu_row(q, inner):
                for u in range(_HH // _LANES):
                    sl = pl.ds(u * _LANES, _LANES)
                    r[p][q, sl] = jnp.maximum(r[p][q, sl], 0.0)
                return inner
            lax.fori_loop(0, _K, relu_row, 0)

        def scatter(p):
            pltpu.sync_copy(r[p], s_sh.at[scat[p]], add=True)

        def process(p, j, jnext):
            q = 1 - p
            bias(q)
            start_a(q)
            start_idx(p, jnext)
            wait_b(p)
            relu(p)
            wait_a(q)
            start_b(q)
            wait_idx(p, jnext)
            scatter(p)

        pltpu.sync_copy(src_h.at[pl.ds(ebase, _K)], ibs[0])
        pltpu.sync_copy(dst_h.at[pl.ds(ebase, _K)], ibd[0])
        bias(0)
        start_a(0)
        pltpu.sync_copy(src_h.at[pl.ds(ebase + _K, _K)], ibs[1])
        pltpu.sync_copy(dst_h.at[pl.ds(ebase + _K, _K)], ibd[1])
        wait_a(0)
        start_b(0)

        last = _NCH - 1

        def pair(m, carry):
            j = 2 * m
            process(0, j, jnp.minimum(j + 2, last))
            process(1, j + 1, jnp.minimum(j + 3, last))
            return carry
        lax.fori_loop(0, _NCH // 2, pair, 0)
        process(0, last, last)
        wait_b(1)

        plsc.subcore_barrier()
        pltpu.sync_copy(s_sh.at[pl.ds(t * _NZ, _NZ)],
                        out_h.at[c, pl.ds(t * _NZ, _NZ)])

        @pl.when(t == _NT - 1)
        def _copy_rem():
            pltpu.sync_copy(s_sh.at[pl.ds(_NT * _NZ, _NREM)],
                            out_h.at[c, pl.ds(_NT * _NZ, _NREM)])

    return sc_edge


_sc_edge = _make_sc_edge_kernel()


def kernel(x, We1, be1, We2, be2, Wn1, bn1, Wn2, bn2, gamma, beta, edge_index):
    L = We1.shape[0]
    assert x.shape == (_N, _H) and edge_index.shape == (2, _E)

    src = edge_index[0]
    dst = edge_index[1]


    counts = jnp.bincount(dst, length=_N).astype(jnp.float32).reshape(_N, 1)

    row = lambda v: v.reshape(1, _H)
    for i in range(L):
        a_fl, b_fl = _pre_projections(x, We1[i], row(be1[i]))
        s = _sc_edge(a_fl, b_fl, src, dst)
        x = _post_update(s[0], s[1], x, counts, We2[i], Wn1[i], Wn2[i],
                         row(be2[i]), row(bn1[i]), row(bn2[i]),
                         row(gamma[i]), row(beta[i]))
    return x
